# initial kernel scaffold (unmeasured)
import jax
import jax.numpy as jnp
from jax import lax
from jax.experimental import pallas as pl
from jax.experimental.pallas import tpu as pltpu

N_DEV = 8
M = 4096
KP = 512
N = 8192
TN = 512
N_TILES = N // TN


def kernel(x, w_mat, scale_x, scale_w):
    x8 = x.astype(jnp.float8_e5m2)
    w8 = w_mat.astype(jnp.float8_e5m2)
    s = (scale_x * scale_w).astype(jnp.float32).reshape(1, 1)

    def body(x_ref, w_ref, s_ref, out_ref, x_all, w_all, sx, rx, sw, rw):
        j = pl.program_id(0)
        my = lax.axis_index("i")
        right = lax.rem(my + 1, N_DEV)
        left = lax.rem(my + N_DEV - 1, N_DEV)

        @pl.when(j == 0)
        def _comm():
            barrier = pltpu.get_barrier_semaphore()
            for nbr in (left, right):
                pl.semaphore_signal(
                    barrier, inc=1, device_id=(nbr,),
                    device_id_type=pl.DeviceIdType.MESH,
                )
            pl.semaphore_wait(barrier, 2)

            x_all[my, :, :] = x_ref[:, :]
            w_all[my, :, :] = w_ref[:, :]

            for h in range(N_DEV - 1):
                c_s = lax.rem(my - h + N_DEV, N_DEV)
                c_r = lax.rem(my - h - 1 + N_DEV, N_DEV)
                send_x = pltpu.make_async_remote_copy(
                    src_ref=x_all.at[c_s], dst_ref=x_all.at[c_s],
                    send_sem=sx.at[c_s], recv_sem=rx.at[c_s],
                    device_id=(right,), device_id_type=pl.DeviceIdType.MESH,
                )
                send_w = pltpu.make_async_remote_copy(
                    src_ref=w_all.at[c_s], dst_ref=w_all.at[c_s],
                    send_sem=sw.at[c_s], recv_sem=rw.at[c_s],
                    device_id=(right,), device_id_type=pl.DeviceIdType.MESH,
                )
                send_x.start()
                send_w.start()
                recv_x = pltpu.make_async_remote_copy(
                    src_ref=x_all.at[c_r], dst_ref=x_all.at[c_r],
                    send_sem=sx.at[c_r], recv_sem=rx.at[c_r],
                    device_id=(right,), device_id_type=pl.DeviceIdType.MESH,
                )
                recv_w = pltpu.make_async_remote_copy(
                    src_ref=w_all.at[c_r], dst_ref=w_all.at[c_r],
                    send_sem=sw.at[c_r], recv_sem=rw.at[c_r],
                    device_id=(right,), device_id_type=pl.DeviceIdType.MESH,
                )
                recv_x.wait_recv()
                recv_w.wait_recv()
                send_x.wait_send()
                send_w.wait_send()

        col0 = j * TN
        acc = jnp.zeros((M, TN), jnp.float32)
        for d in range(N_DEV):
            acc = acc + lax.dot_general(
                x_all[d, :, :],
                w_all[d, :, pl.ds(col0, TN)],
                (((1,), (0,)), ((), ())),
                preferred_element_type=jnp.float32,
            )
        out_ref[:, :] = jnp.maximum(acc * s_ref[0, 0], 0.0)

    return pl.pallas_call(
        body,
        grid=(N_TILES,),
        out_shape=jax.ShapeDtypeStruct((M, N), jnp.float32),
        in_specs=[
            pl.BlockSpec((M, KP), lambda j: (0, 0)),
            pl.BlockSpec((KP, N), lambda j: (0, 0)),
            pl.BlockSpec(memory_space=pltpu.SMEM),
        ],
        out_specs=pl.BlockSpec((M, TN), lambda j: (0, j)),
        scratch_shapes=[
            pltpu.VMEM((N_DEV, M, KP), jnp.float8_e5m2),
            pltpu.VMEM((N_DEV, KP, N), jnp.float8_e5m2),
            pltpu.SemaphoreType.DMA((N_DEV,)),
            pltpu.SemaphoreType.DMA((N_DEV,)),
            pltpu.SemaphoreType.DMA((N_DEV,)),
            pltpu.SemaphoreType.DMA((N_DEV,)),
        ],
        compiler_params=pltpu.CompilerParams(collective_id=0),
    )(x8, w8, s)


# baseline (device time: 742110 ns/iter reference)
import jax
import jax.numpy as jnp
from jax import lax
from jax.experimental import pallas as pl
from jax.experimental.pallas import tpu as pltpu

N_DEV = 8
M = 4096
KP = 512
N = 8192
TN = 256
N_TILES = N // TN


def kernel(x, w_mat, scale_x, scale_w):
    x8 = x.astype(jnp.float8_e5m2)
    w8 = w_mat.astype(jnp.float8_e5m2)
    s = (scale_x * scale_w).astype(jnp.float32).reshape(1, 1)

    def body(x_ref, w_ref, s_ref, out_ref, x_all, w_all, sx, rx, sw, rw, cx, cw):
        j = pl.program_id(0)
        my = lax.axis_index("i")
        right = lax.rem(my + 1, N_DEV)
        left = lax.rem(my + N_DEV - 1, N_DEV)

        @pl.when(j == 0)
        def _comm():
            barrier = pltpu.get_barrier_semaphore()
            for nbr in (left, right):
                pl.semaphore_signal(
                    barrier, inc=1, device_id=(nbr,),
                    device_id_type=pl.DeviceIdType.MESH,
                )
            pl.semaphore_wait(barrier, 2)

            copy_x = pltpu.make_async_copy(x_ref, x_all.at[my], cx)
            copy_w = pltpu.make_async_copy(w_ref, w_all.at[my], cw)
            copy_x.start()
            copy_w.start()
            copy_x.wait()
            copy_w.wait()

            for h in range(N_DEV - 1):
                c_s = lax.rem(my - h + N_DEV, N_DEV)
                c_r = lax.rem(my - h - 1 + N_DEV, N_DEV)
                send_x = pltpu.make_async_remote_copy(
                    src_ref=x_all.at[c_s], dst_ref=x_all.at[c_s],
                    send_sem=sx.at[c_s], recv_sem=rx.at[c_s],
                    device_id=(right,), device_id_type=pl.DeviceIdType.MESH,
                )
                send_w = pltpu.make_async_remote_copy(
                    src_ref=w_all.at[c_s], dst_ref=w_all.at[c_s],
                    send_sem=sw.at[c_s], recv_sem=rw.at[c_s],
                    device_id=(right,), device_id_type=pl.DeviceIdType.MESH,
                )
                send_x.start()
                send_w.start()
                recv_x = pltpu.make_async_remote_copy(
                    src_ref=x_all.at[c_r], dst_ref=x_all.at[c_r],
                    send_sem=sx.at[c_r], recv_sem=rx.at[c_r],
                    device_id=(right,), device_id_type=pl.DeviceIdType.MESH,
                )
                recv_w = pltpu.make_async_remote_copy(
                    src_ref=w_all.at[c_r], dst_ref=w_all.at[c_r],
                    send_sem=sw.at[c_r], recv_sem=rw.at[c_r],
                    device_id=(right,), device_id_type=pl.DeviceIdType.MESH,
                )
                recv_x.wait_recv()
                recv_w.wait_recv()
                send_x.wait_send()
                send_w.wait_send()

        col0 = j * TN
        TM = 1024
        for mi in range(M // TM):
            acc = jnp.zeros((TM, TN), jnp.float32)
            for d in range(N_DEV):
                acc = acc + lax.dot_general(
                    x_all[d, pl.ds(mi * TM, TM), :],
                    w_all[d, :, pl.ds(col0, TN)],
                    (((1,), (0,)), ((), ())),
                    preferred_element_type=jnp.float32,
                )
            out_ref[pl.ds(mi * TM, TM), :] = jnp.maximum(acc * s_ref[0, 0], 0.0)

    return pl.pallas_call(
        body,
        grid=(N_TILES,),
        out_shape=jax.ShapeDtypeStruct((M, N), jnp.float32),
        in_specs=[
            pl.BlockSpec(memory_space=pltpu.MemorySpace.HBM),
            pl.BlockSpec(memory_space=pltpu.MemorySpace.HBM),
            pl.BlockSpec(memory_space=pltpu.SMEM),
        ],
        out_specs=pl.BlockSpec((M, TN), lambda j: (0, j)),
        scratch_shapes=[
            pltpu.VMEM((N_DEV, M, KP), jnp.float8_e5m2),
            pltpu.VMEM((N_DEV, KP, N), jnp.float8_e5m2),
            pltpu.SemaphoreType.DMA((N_DEV,)),
            pltpu.SemaphoreType.DMA((N_DEV,)),
            pltpu.SemaphoreType.DMA((N_DEV,)),
            pltpu.SemaphoreType.DMA((N_DEV,)),
            pltpu.SemaphoreType.DMA,
            pltpu.SemaphoreType.DMA,
        ],
        compiler_params=pltpu.CompilerParams(
            collective_id=0,
            vmem_limit_bytes=63 * 1024 * 1024,
        ),
    )(x8, w8, s)


# device time: 536844 ns/iter; 1.3824x vs baseline; 1.3824x over previous
import jax
import jax.numpy as jnp
from jax import lax
from jax.experimental import pallas as pl
from jax.experimental.pallas import tpu as pltpu

N_DEV = 8
M = 4096
KP = 512
N = 8192
TN = 256
N_TILES = N // TN


def kernel(x, w_mat, scale_x, scale_w):
    x8 = x.astype(jnp.float8_e5m2)
    w8 = w_mat.astype(jnp.float8_e5m2)
    s = (scale_x * scale_w).astype(jnp.float32).reshape(1, 1)

    def body(x_ref, w_ref, s_ref, out_ref, x_all, w_all,
             sxf, rxf, swf, rwf, sxb, rxb, swb, rwb, cx, cw):
        j = pl.program_id(0)
        my = lax.axis_index("i")
        right = lax.rem(my + 1, N_DEV)
        left = lax.rem(my + N_DEV - 1, N_DEV)

        @pl.when(j == 0)
        def _comm():
            barrier = pltpu.get_barrier_semaphore()
            for nbr in (left, right):
                pl.semaphore_signal(
                    barrier, inc=1, device_id=(nbr,),
                    device_id_type=pl.DeviceIdType.MESH,
                )
            pl.semaphore_wait(barrier, 2)

            copy_x = pltpu.make_async_copy(x_ref, x_all.at[my], cx)
            copy_w = pltpu.make_async_copy(w_ref, w_all.at[my], cw)
            copy_x.start()
            copy_w.start()
            copy_x.wait()
            copy_w.wait()

            def _copy(tensor_all, sem_arr_s, sem_arr_r, c, dev):
                return pltpu.make_async_remote_copy(
                    src_ref=tensor_all.at[c], dst_ref=tensor_all.at[c],
                    send_sem=sem_arr_s.at[c], recv_sem=sem_arr_r.at[c],
                    device_id=(dev,), device_id_type=pl.DeviceIdType.MESH,
                )

            FWD = N_DEV // 2
            BWD = N_DEV - 1 - FWD
            for r in range(FWD):
                f_s = lax.rem(my - r + N_DEV, N_DEV)
                f_r = lax.rem(my - r - 1 + N_DEV, N_DEV)
                sends = [
                    _copy(x_all, sxf, rxf, f_s, right),
                    _copy(w_all, swf, rwf, f_s, right),
                ]
                recvs = [
                    _copy(x_all, sxf, rxf, f_r, right),
                    _copy(w_all, swf, rwf, f_r, right),
                ]
                if r < BWD:
                    b_s = lax.rem(my + r, N_DEV)
                    b_r = lax.rem(my + r + 1, N_DEV)
                    sends += [
                        _copy(x_all, sxb, rxb, b_s, left),
                        _copy(w_all, swb, rwb, b_s, left),
                    ]
                    recvs += [
                        _copy(x_all, sxb, rxb, b_r, left),
                        _copy(w_all, swb, rwb, b_r, left),
                    ]
                for s_ in sends:
                    s_.start()
                for r_ in recvs:
                    r_.wait_recv()
                for s_ in sends:
                    s_.wait_send()

        col0 = j * TN
        TM = 1024
        for mi in range(M // TM):
            acc = jnp.zeros((TM, TN), jnp.float32)
            for d in range(N_DEV):
                acc = acc + lax.dot_general(
                    x_all[d, pl.ds(mi * TM, TM), :],
                    w_all[d, :, pl.ds(col0, TN)],
                    (((1,), (0,)), ((), ())),
                    preferred_element_type=jnp.float32,
                )
            out_ref[pl.ds(mi * TM, TM), :] = jnp.maximum(acc * s_ref[0, 0], 0.0)

    return pl.pallas_call(
        body,
        grid=(N_TILES,),
        out_shape=jax.ShapeDtypeStruct((M, N), jnp.float32),
        in_specs=[
            pl.BlockSpec(memory_space=pltpu.MemorySpace.HBM),
            pl.BlockSpec(memory_space=pltpu.MemorySpace.HBM),
            pl.BlockSpec(memory_space=pltpu.SMEM),
        ],
        out_specs=pl.BlockSpec((M, TN), lambda j: (0, j)),
        scratch_shapes=[
            pltpu.VMEM((N_DEV, M, KP), jnp.float8_e5m2),
            pltpu.VMEM((N_DEV, KP, N), jnp.float8_e5m2),
            pltpu.SemaphoreType.DMA((N_DEV,)),
            pltpu.SemaphoreType.DMA((N_DEV,)),
            pltpu.SemaphoreType.DMA((N_DEV,)),
            pltpu.SemaphoreType.DMA((N_DEV,)),
            pltpu.SemaphoreType.DMA((N_DEV,)),
            pltpu.SemaphoreType.DMA((N_DEV,)),
            pltpu.SemaphoreType.DMA((N_DEV,)),
            pltpu.SemaphoreType.DMA((N_DEV,)),
            pltpu.SemaphoreType.DMA,
            pltpu.SemaphoreType.DMA,
        ],
        compiler_params=pltpu.CompilerParams(
            collective_id=0,
            vmem_limit_bytes=63 * 1024 * 1024,
        ),
    )(x8, w8, s)


# device time: 536745 ns/iter; 1.3826x vs baseline; 1.0002x over previous
import jax
import jax.numpy as jnp
from jax import lax
from jax.experimental import pallas as pl
from jax.experimental.pallas import tpu as pltpu

N_DEV = 8
M = 4096
KP = 512
N = 8192
TN = 256
TM = 1024
MB = M // TM
NB = (N // TN) * MB
DEPTH = 4
FWD = N_DEV // 2
BWD = N_DEV - 1 - FWD


def kernel(x, w_mat, scale_x, scale_w):
    x8 = x.astype(jnp.float8_e5m2)
    w8 = w_mat.astype(jnp.float8_e5m2)
    s = (scale_x * scale_w).astype(jnp.float32).reshape(1, 1)

    def body(x_ref, w_ref, s_ref, out_ref, x_all, w_all,
             sxf, rxf, swf, rwf, sxb, rxb, swb, rwb, cx, cw,
             pin, stage, pin_sem, so_sem):
        my = lax.axis_index("i")
        right = lax.rem(my + 1, N_DEV)
        left = lax.rem(my + N_DEV - 1, N_DEV)

        def blk_rc(i):
            return lax.rem(i, MB) * TM, (i // MB) * TN

        def pin_copy(i, slot):
            r0, c0 = blk_rc(i)
            return pltpu.make_async_copy(
                out_ref.at[pl.ds(r0, TM), pl.ds(c0, TN)],
                pin.at[slot], pin_sem.at[slot])

        def stage_copy(i, slot):
            r0, c0 = blk_rc(i)
            return pltpu.make_async_copy(
                stage.at[slot],
                out_ref.at[pl.ds(r0, TM), pl.ds(c0, TN)], so_sem.at[slot])

        def run_pass(chunks, first, last):
            if not first:
                for i in range(DEPTH):
                    pin_copy(i, i).start()

            def iter_body(i, _):
                slot = lax.rem(i, DEPTH)
                r0, c0 = blk_rc(i)
                acc = jnp.zeros((TM, TN), jnp.float32)
                for d in chunks:
                    acc = acc + lax.dot_general(
                        x_all[d, pl.ds(r0, TM), :],
                        w_all[d, :, pl.ds(c0, TN)],
                        (((1,), (0,)), ((), ())),
                        preferred_element_type=jnp.float32,
                    )
                if not first:
                    pin_copy(i, slot).wait()
                    acc = acc + pin[slot]
                    @pl.when(i + DEPTH < NB)
                    def _():
                        pin_copy(i + DEPTH, slot).start()
                if last:
                    acc = jnp.maximum(acc * s_ref[0, 0], 0.0)

                @pl.when(i >= DEPTH)
                def _():
                    stage_copy(i - DEPTH, slot).wait()
                stage[slot] = acc
                stage_copy(i, slot).start()
                return 0

            lax.fori_loop(0, NB, iter_body, 0)
            for i in range(NB - DEPTH, NB):
                stage_copy(i, i % DEPTH).wait()

        barrier = pltpu.get_barrier_semaphore()
        for nbr in (left, right):
            pl.semaphore_signal(
                barrier, inc=1, device_id=(nbr,),
                device_id_type=pl.DeviceIdType.MESH,
            )
        pl.semaphore_wait(barrier, 2)

        copy_x = pltpu.make_async_copy(x_ref, x_all.at[my], cx)
        copy_w = pltpu.make_async_copy(w_ref, w_all.at[my], cw)
        copy_x.start()
        copy_w.start()
        copy_x.wait()
        copy_w.wait()

        def _rcopy(tensor_all, sem_s, sem_r, c, dev):
            return pltpu.make_async_remote_copy(
                src_ref=tensor_all.at[c], dst_ref=tensor_all.at[c],
                send_sem=sem_s.at[c], recv_sem=sem_r.at[c],
                device_id=(dev,), device_id_type=pl.DeviceIdType.MESH,
            )

        fwd = [lax.rem(my - k + N_DEV, N_DEV) for k in range(1, FWD + 1)]
        bwd = [lax.rem(my + k, N_DEV) for k in range(1, BWD + 1)]
        passes = [[my]] + [
            [fwd[k]] + ([bwd[k]] if k < BWD else []) for k in range(FWD)
        ]

        for r in range(FWD):
            f_s = lax.rem(my - r + N_DEV, N_DEV)
            sends = [
                _rcopy(x_all, sxf, rxf, f_s, right),
                _rcopy(w_all, swf, rwf, f_s, right),
            ]
            recvs = [
                _rcopy(x_all, sxf, rxf, fwd[r], right),
                _rcopy(w_all, swf, rwf, fwd[r], right),
            ]
            if r < BWD:
                b_s = lax.rem(my + r, N_DEV)
                sends += [
                    _rcopy(x_all, sxb, rxb, b_s, left),
                    _rcopy(w_all, swb, rwb, b_s, left),
                ]
                recvs += [
                    _rcopy(x_all, sxb, rxb, bwd[r], left),
                    _rcopy(w_all, swb, rwb, bwd[r], left),
                ]
            for s_ in sends:
                s_.start()
            run_pass(passes[r], first=(r == 0), last=False)
            for r_ in recvs:
                r_.wait_recv()
            for s_ in sends:
                s_.wait_send()

        run_pass(passes[FWD], first=False, last=True)

    return pl.pallas_call(
        body,
        out_shape=jax.ShapeDtypeStruct((M, N), jnp.float32),
        in_specs=[
            pl.BlockSpec(memory_space=pltpu.MemorySpace.HBM),
            pl.BlockSpec(memory_space=pltpu.MemorySpace.HBM),
            pl.BlockSpec(memory_space=pltpu.SMEM),
        ],
        out_specs=pl.BlockSpec(memory_space=pltpu.MemorySpace.HBM),
        scratch_shapes=[
            pltpu.VMEM((N_DEV, M, KP), jnp.float8_e5m2),
            pltpu.VMEM((N_DEV, KP, N), jnp.float8_e5m2),
            pltpu.SemaphoreType.DMA((N_DEV,)),
            pltpu.SemaphoreType.DMA((N_DEV,)),
            pltpu.SemaphoreType.DMA((N_DEV,)),
            pltpu.SemaphoreType.DMA((N_DEV,)),
            pltpu.SemaphoreType.DMA((N_DEV,)),
            pltpu.SemaphoreType.DMA((N_DEV,)),
            pltpu.SemaphoreType.DMA((N_DEV,)),
            pltpu.SemaphoreType.DMA((N_DEV,)),
            pltpu.SemaphoreType.DMA,
            pltpu.SemaphoreType.DMA,
            pltpu.VMEM((DEPTH, TM, TN), jnp.float32),
            pltpu.VMEM((DEPTH, TM, TN), jnp.float32),
            pltpu.SemaphoreType.DMA((DEPTH,)),
            pltpu.SemaphoreType.DMA((DEPTH,)),
        ],
        compiler_params=pltpu.CompilerParams(
            collective_id=0,
            vmem_limit_bytes=63 * 1024 * 1024,
        ),
    )(x8, w8, s)


# device time: 518156 ns/iter; 1.4322x vs baseline; 1.0359x over previous
import jax
import jax.numpy as jnp
from jax import lax
from jax.experimental import pallas as pl
from jax.experimental.pallas import tpu as pltpu

N_DEV = 8
M = 4096
KP = 512
N = 8192
TN = 256
TM = 1024
MB = M // TM
NB = (N // TN) * MB
NB2 = 72
DEPTH = 4
FWD = N_DEV // 2
BWD = N_DEV - 1 - FWD


def kernel(x, w_mat, scale_x, scale_w):
    x8 = x.astype(jnp.float8_e5m2)
    w8 = w_mat.astype(jnp.float8_e5m2)
    s = (scale_x * scale_w).astype(jnp.float32).reshape(1, 1)

    def body(x_ref, w_ref, s_ref, out_ref, x_all, w_all,
             sxf, rxf, swf, rwf, sxb, rxb, swb, rwb, cx, cw,
             pin, stage, pin_sem, so_sem):
        my = lax.axis_index("i")
        right = lax.rem(my + 1, N_DEV)
        left = lax.rem(my + N_DEV - 1, N_DEV)

        def blk_rc(i):
            return lax.rem(i, MB) * TM, (i // MB) * TN

        def pin_copy(i, slot):
            r0, c0 = blk_rc(i)
            return pltpu.make_async_copy(
                out_ref.at[pl.ds(r0, TM), pl.ds(c0, TN)],
                pin.at[slot], pin_sem.at[slot])

        def stage_copy(i, slot):
            r0, c0 = blk_rc(i)
            return pltpu.make_async_copy(
                stage.at[slot],
                out_ref.at[pl.ds(r0, TM), pl.ds(c0, TN)], so_sem.at[slot])

        def run_pass(chunks, first, last, lo, hi):
            if not first:
                for i in range(lo, lo + DEPTH):
                    pin_copy(i, i % DEPTH).start()

            def iter_body(i, _):
                slot = lax.rem(i, DEPTH)
                r0, c0 = blk_rc(i)
                acc = jnp.zeros((TM, TN), jnp.float32)
                for d in chunks:
                    acc = acc + lax.dot_general(
                        x_all[d, pl.ds(r0, TM), :],
                        w_all[d, :, pl.ds(c0, TN)],
                        (((1,), (0,)), ((), ())),
                        preferred_element_type=jnp.float32,
                    )
                if not first:
                    pin_copy(i, slot).wait()
                    acc = acc + pin[slot]
                    @pl.when(i + DEPTH < hi)
                    def _():
                        pin_copy(i + DEPTH, slot).start()
                if last:
                    acc = jnp.maximum(acc * s_ref[0, 0], 0.0)

                @pl.when(i >= lo + DEPTH)
                def _():
                    stage_copy(i - DEPTH, slot).wait()
                stage[slot] = acc
                stage_copy(i, slot).start()
                return 0

            lax.fori_loop(lo, hi, iter_body, 0)
            for i in range(hi - DEPTH, hi):
                stage_copy(i, i % DEPTH).wait()

        barrier = pltpu.get_barrier_semaphore()
        for nbr in (left, right):
            pl.semaphore_signal(
                barrier, inc=1, device_id=(nbr,),
                device_id_type=pl.DeviceIdType.MESH,
            )
        pl.semaphore_wait(barrier, 2)

        copy_x = pltpu.make_async_copy(x_ref, x_all.at[my], cx)
        copy_w = pltpu.make_async_copy(w_ref, w_all.at[my], cw)
        copy_x.start()
        copy_w.start()
        copy_x.wait()
        copy_w.wait()

        def _rcopy(tensor_all, sem_s, sem_r, c, dev):
            return pltpu.make_async_remote_copy(
                src_ref=tensor_all.at[c], dst_ref=tensor_all.at[c],
                send_sem=sem_s.at[c], recv_sem=sem_r.at[c],
                device_id=(dev,), device_id_type=pl.DeviceIdType.MESH,
            )

        fwd = [lax.rem(my - k + N_DEV, N_DEV) for k in range(1, FWD + 1)]
        bwd = [lax.rem(my + k, N_DEV) for k in range(1, BWD + 1)]
        passes = [[my]] + [
            [fwd[k]] + ([bwd[k]] if k < BWD else []) for k in range(FWD)
        ]

        for r in range(FWD):
            f_s = lax.rem(my - r + N_DEV, N_DEV)
            sends = [
                _rcopy(x_all, sxf, rxf, f_s, right),
                _rcopy(w_all, swf, rwf, f_s, right),
            ]
            recvs = [
                _rcopy(x_all, sxf, rxf, fwd[r], right),
                _rcopy(w_all, swf, rwf, fwd[r], right),
            ]
            if r < BWD:
                b_s = lax.rem(my + r, N_DEV)
                sends += [
                    _rcopy(x_all, sxb, rxb, b_s, left),
                    _rcopy(w_all, swb, rwb, b_s, left),
                ]
                recvs += [
                    _rcopy(x_all, sxb, rxb, bwd[r], left),
                    _rcopy(w_all, swb, rwb, bwd[r], left),
                ]
            for s_ in sends:
                s_.start()
            run_pass(passes[r], first=(r == 0), last=False, lo=0, hi=NB2)
            for r_ in recvs:
                r_.wait_recv()
            for s_ in sends:
                s_.wait_send()

        run_pass(passes[FWD], first=False, last=True, lo=0, hi=NB2)
        all_chunks = [my] + fwd + bwd
        run_pass(all_chunks, first=True, last=True, lo=NB2, hi=NB)

    return pl.pallas_call(
        body,
        out_shape=jax.ShapeDtypeStruct((M, N), jnp.float32),
        in_specs=[
            pl.BlockSpec(memory_space=pltpu.MemorySpace.HBM),
            pl.BlockSpec(memory_space=pltpu.MemorySpace.HBM),
            pl.BlockSpec(memory_space=pltpu.SMEM),
        ],
        out_specs=pl.BlockSpec(memory_space=pltpu.MemorySpace.HBM),
        scratch_shapes=[
            pltpu.VMEM((N_DEV, M, KP), jnp.float8_e5m2),
            pltpu.VMEM((N_DEV, KP, N), jnp.float8_e5m2),
            pltpu.SemaphoreType.DMA((N_DEV,)),
            pltpu.SemaphoreType.DMA((N_DEV,)),
            pltpu.SemaphoreType.DMA((N_DEV,)),
            pltpu.SemaphoreType.DMA((N_DEV,)),
            pltpu.SemaphoreType.DMA((N_DEV,)),
            pltpu.SemaphoreType.DMA((N_DEV,)),
            pltpu.SemaphoreType.DMA((N_DEV,)),
            pltpu.SemaphoreType.DMA((N_DEV,)),
            pltpu.SemaphoreType.DMA,
            pltpu.SemaphoreType.DMA,
            pltpu.VMEM((DEPTH, TM, TN), jnp.float32),
            pltpu.VMEM((DEPTH, TM, TN), jnp.float32),
            pltpu.SemaphoreType.DMA((DEPTH,)),
            pltpu.SemaphoreType.DMA((DEPTH,)),
        ],
        compiler_params=pltpu.CompilerParams(
            collective_id=0,
            vmem_limit_bytes=63 * 1024 * 1024,
        ),
    )(x8, w8, s)


# device time: 422195 ns/iter; 1.7577x vs baseline; 1.2273x over previous
import jax
import jax.numpy as jnp
from jax import lax
from jax.experimental import pallas as pl
from jax.experimental.pallas import tpu as pltpu

N_DEV = 8
M = 4096
KP = 512
N = 8192
TN = 256
TM = 1024
MB = M // TM
NB = (N // TN) * MB
SEG = 72
DEPTH = 4
MH = M // 2
WH = KP // 4


def kernel(x, w_mat, scale_x, scale_w):
    x8 = x.astype(jnp.float8_e5m2)
    w8 = w_mat.astype(jnp.float8_e5m2)
    s = (scale_x * scale_w).astype(jnp.float32).reshape(1, 1)

    def body(x_ref, w_ref, s_ref, out_ref, x_all, w_all,
             sx, sw, rx, rw, s3, r3, cx, cw,
             pin, stage, pin_sem, so_sem):
        my = lax.axis_index("i")

        def from_xyz(xx, yy, zz):
            return 4 * zz + 2 * yy + (xx + yy - 2 * xx * yy)

        z = my // 4
        p = lax.rem(my, 4)
        y = p // 2
        px = lax.rem(p, 2)
        x_c = px + y - 2 * px * y
        nbx = from_xyz(1 - x_c, y, z)
        nby = from_xyz(x_c, 1 - y, z)
        nbz = from_xyz(x_c, y, 1 - z)
        dxy = from_xyz(1 - x_c, 1 - y, z)
        dxz = from_xyz(1 - x_c, y, 1 - z)
        dyz = from_xyz(x_c, 1 - y, 1 - z)
        ant = from_xyz(1 - x_c, 1 - y, 1 - z)
        nbrs = [nbx, nby, nbz]

        def blk_rc(i):
            return lax.rem(i, MB) * TM, (i // MB) * TN

        def pin_copy(i, slot):
            r0, c0 = blk_rc(i)
            return pltpu.make_async_copy(
                out_ref.at[pl.ds(r0, TM), pl.ds(c0, TN)],
                pin.at[slot], pin_sem.at[slot])

        def stage_copy(i, slot):
            r0, c0 = blk_rc(i)
            return pltpu.make_async_copy(
                stage.at[slot],
                out_ref.at[pl.ds(r0, TM), pl.ds(c0, TN)], so_sem.at[slot])

        def run_pass(chunks, first, last, lo, hi):
            if not first:
                for i in range(lo, lo + DEPTH):
                    pin_copy(i, i % DEPTH).start()

            def iter_body(i, _):
                slot = lax.rem(i, DEPTH)
                r0, c0 = blk_rc(i)
                acc = jnp.zeros((TM, TN), jnp.float32)
                for d in chunks:
                    acc = acc + lax.dot_general(
                        x_all[d, pl.ds(r0, TM), :],
                        w_all[d, :, pl.ds(c0, TN)],
                        (((1,), (0,)), ((), ())),
                        preferred_element_type=jnp.float32,
                    )
                if not first:
                    pin_copy(i, slot).wait()
                    acc = acc + pin[slot]
                    @pl.when(i + DEPTH < hi)
                    def _():
                        pin_copy(i + DEPTH, slot).start()
                if last:
                    acc = jnp.maximum(acc * s_ref[0, 0], 0.0)

                @pl.when(i >= lo + DEPTH)
                def _():
                    stage_copy(i - DEPTH, slot).wait()
                stage[slot] = acc
                stage_copy(i, slot).start()
                return 0

            lax.fori_loop(lo, hi, iter_body, 0)
            for i in range(hi - DEPTH, hi):
                stage_copy(i, i % DEPTH).wait()

        barrier = pltpu.get_barrier_semaphore()
        for nbr in nbrs:
            pl.semaphore_signal(
                barrier, inc=1, device_id=(nbr,),
                device_id_type=pl.DeviceIdType.MESH,
            )
        pl.semaphore_wait(barrier, 3)

        copy_x = pltpu.make_async_copy(x_ref, x_all.at[my], cx)
        copy_w = pltpu.make_async_copy(w_ref, w_all.at[my], cw)
        copy_x.start()
        copy_w.start()
        copy_x.wait()
        copy_w.wait()

        def remote(src, dst, ssem, rsem, dev):
            return pltpu.make_async_remote_copy(
                src_ref=src, dst_ref=dst, send_sem=ssem, recv_sem=rsem,
                device_id=(dev,), device_id_type=pl.DeviceIdType.MESH,
            )

        def full_chunk(c, ssem_i, dev):
            return [
                remote(x_all.at[c], x_all.at[c], sx.at[ssem_i], rx.at[c], dev),
                remote(w_all.at[c], w_all.at[c], sw.at[ssem_i], rw.at[c], dev),
            ]

        def full_chunk_recv(c):
            return full_chunk(c, 6, nbx)

        r1_sends = []
        for k, nbr in enumerate(nbrs):
            r1_sends += full_chunk(my, k, nbr)
        for t in r1_sends:
            t.start()
        r1_recvs = []
        for nbr in nbrs:
            r1_recvs += full_chunk_recv(nbr)
        for t in r1_recvs:
            t.wait_recv()
        for t in r1_sends:
            t.wait_send()

        r2_sends = []
        for k, (c, nbr) in enumerate(zip((nby, nbz, nbx), nbrs)):
            r2_sends += full_chunk(c, 3 + k, nbr)
        for t in r2_sends:
            t.start()

        run_pass([my, nbx, nby, nbz], first=True, last=False, lo=0, hi=SEG)

        r2_recvs = []
        for c in (dxy, dyz, dxz):
            r2_recvs += full_chunk_recv(c)
        for t in r2_recvs:
            t.wait_recv()
        for t in r2_sends:
            t.wait_send()

        def r3_descs(cx_, cy_, cz_, s0, d0, d1, d2):
            return [
                remote(x_all.at[cx_, pl.ds(0, MH), :],
                       x_all.at[cx_, pl.ds(0, MH), :],
                       s3.at[s0 + 0], r3.at[0], d0),
                remote(w_all.at[cx_, pl.ds(0, WH), :],
                       w_all.at[cx_, pl.ds(0, WH), :],
                       s3.at[s0 + 1], r3.at[1], d0),
                remote(x_all.at[cy_, pl.ds(MH, MH), :],
                       x_all.at[cy_, pl.ds(MH, MH), :],
                       s3.at[s0 + 2], r3.at[2], d1),
                remote(w_all.at[cy_, pl.ds(WH, WH), :],
                       w_all.at[cy_, pl.ds(WH, WH), :],
                       s3.at[s0 + 3], r3.at[3], d1),
                remote(w_all.at[cz_, pl.ds(2 * WH, 2 * WH), :],
                       w_all.at[cz_, pl.ds(2 * WH, 2 * WH), :],
                       s3.at[s0 + 4], r3.at[4], d2),
            ]

        r3_sends = r3_descs(dyz, dxz, dxy, 0, nbx, nby, nbz)
        for t in r3_sends:
            t.start()

        run_pass([my, nbx, nby, nbz], first=True, last=False, lo=SEG, hi=NB)

        r3_recvs = r3_descs(ant, ant, ant, 5, nbx, nbx, nbx)
        for t in r3_recvs:
            t.wait_recv()
        for t in r3_sends:
            t.wait_send()

        run_pass([dxy, dxz, dyz, ant], first=False, last=True, lo=0, hi=NB)

    return pl.pallas_call(
        body,
        out_shape=jax.ShapeDtypeStruct((M, N), jnp.float32),
        in_specs=[
            pl.BlockSpec(memory_space=pltpu.MemorySpace.HBM),
            pl.BlockSpec(memory_space=pltpu.MemorySpace.HBM),
            pl.BlockSpec(memory_space=pltpu.SMEM),
        ],
        out_specs=pl.BlockSpec(memory_space=pltpu.MemorySpace.HBM),
        scratch_shapes=[
            pltpu.VMEM((N_DEV, M, KP), jnp.float8_e5m2),
            pltpu.VMEM((N_DEV, KP, N), jnp.float8_e5m2),
            pltpu.SemaphoreType.DMA((7,)),
            pltpu.SemaphoreType.DMA((7,)),
            pltpu.SemaphoreType.DMA((N_DEV,)),
            pltpu.SemaphoreType.DMA((N_DEV,)),
            pltpu.SemaphoreType.DMA((10,)),
            pltpu.SemaphoreType.DMA((5,)),
            pltpu.SemaphoreType.DMA,
            pltpu.SemaphoreType.DMA,
            pltpu.VMEM((DEPTH, TM, TN), jnp.float32),
            pltpu.VMEM((DEPTH, TM, TN), jnp.float32),
            pltpu.SemaphoreType.DMA((DEPTH,)),
            pltpu.SemaphoreType.DMA((DEPTH,)),
        ],
        compiler_params=pltpu.CompilerParams(
            collective_id=0,
            vmem_limit_bytes=63 * 1024 * 1024,
        ),
    )(x8, w8, s)
